# baseline (device time: 665464 ns/iter reference)
import jax
import jax.numpy as jnp
from jax import lax
from jax.experimental import pallas as pl
from jax.experimental.pallas import tpu as pltpu

N_DEV = 4
N_CHUNK = 4
BM = 1024


def _make_fused_body(m, k_sh, n):
    h = k_sh // 2
    cw_ = n // N_CHUNK
    sub_per_chunk = m // BM

    def body(x_ref, w_ref, s_ref, y_ref, wg_ref, xg_ref,
             xv, wv, yv, local_sems, wv_sems, yv_sems,
             sx_sems, rx_sems, sw_sems, rw_sems):
        me = lax.axis_index("i")
        a = jnp.bitwise_xor(me, 1)
        b = 3 - me

        def start(src, dst, ssem, rsem, dev):
            c = pltpu.make_async_remote_copy(
                src_ref=src, dst_ref=dst, send_sem=ssem, recv_sem=rsem,
                device_id=(dev,), device_id_type=pl.DeviceIdType.MESH)
            c.start()
            return c

        def recv_wait(dst, dummy_src, rsem):
            pltpu.make_async_remote_copy(
                src_ref=dummy_src, dst_ref=dst,
                send_sem=rsem, recv_sem=rsem,
                device_id=(me,),
                device_id_type=pl.DeviceIdType.MESH).wait_recv()

        def x_slot(q, half):
            return xg_ref.at[:, pl.ds(q * k_sh + half * h, h)]

        def x_half(half):
            return x_ref.at[:, pl.ds(half * h, h)]

        def w_slot(q, half, c):
            return wg_ref.at[pl.ds(q * k_sh + half * h, h),
                             pl.ds(c * cw_, cw_)]

        def w_half(half, c):
            return w_ref.at[pl.ds(half * h, h), pl.ds(c * cw_, cw_)]

        barrier = pltpu.get_barrier_semaphore()
        for p in (a, b):
            pl.semaphore_signal(barrier, inc=1, device_id=(p,),
                                device_id_type=pl.DeviceIdType.MESH)
        pl.semaphore_wait(barrier, 2)

        cx = pltpu.make_async_copy(
            x_ref, xg_ref.at[:, pl.ds(me * k_sh, k_sh)], local_sems.at[0])
        cw = pltpu.make_async_copy(
            w_ref, wg_ref.at[pl.ds(me * k_sh, k_sh), :], local_sems.at[1])
        cx.start()
        cw.start()

        sends = []

        sends.append(start(x_half(0), x_slot(me, 0), sx_sems.at[0], rx_sems.at[0], a))
        sends.append(start(x_half(1), x_slot(me, 1), sx_sems.at[1], rx_sems.at[1], b))
        sends.append(start(x_half(0), x_slot(me, 0), sx_sems.at[2], rx_sems.at[2], b))
        sends.append(start(x_half(1), x_slot(me, 1), sx_sems.at[4], rx_sems.at[4], a))
        recv_wait(x_slot(a, 0), x_half(0), rx_sems.at[0])
        sends.append(start(x_slot(a, 0), x_slot(a, 0), sx_sems.at[3], rx_sems.at[3], b))
        recv_wait(x_slot(b, 1), x_half(1), rx_sems.at[1])
        sends.append(start(x_slot(b, 1), x_slot(b, 1), sx_sems.at[5], rx_sems.at[5], a))

        def w_imm(c):
            sends.append(start(w_half(0, c), w_slot(me, 0, c), sw_sems.at[c, 0], rw_sems.at[c, 0], a))
            sends.append(start(w_half(1, c), w_slot(me, 1, c), sw_sems.at[c, 1], rw_sems.at[c, 1], b))
            sends.append(start(w_half(0, c), w_slot(me, 0, c), sw_sems.at[c, 2], rw_sems.at[c, 2], b))
            sends.append(start(w_half(1, c), w_slot(me, 1, c), sw_sems.at[c, 4], rw_sems.at[c, 4], a))

        def w_fwd(c):
            recv_wait(w_slot(a, 0, c), w_half(0, c), rw_sems.at[c, 0])
            sends.append(start(w_slot(a, 0, c), w_slot(a, 0, c),
                               sw_sems.at[c, 3], rw_sems.at[c, 3], b))
            recv_wait(w_slot(b, 1, c), w_half(1, c), rw_sems.at[c, 1])
            sends.append(start(w_slot(b, 1, c), w_slot(b, 1, c),
                               sw_sems.at[c, 5], rw_sems.at[c, 5], a))

        def w_complete(c):
            recv_wait(w_slot(b, 0, c), w_half(0, c), rw_sems.at[c, 2])
            recv_wait(w_slot(jnp.bitwise_xor(b, 1), 0, c), w_half(0, c), rw_sems.at[c, 3])
            recv_wait(w_slot(a, 1, c), w_half(1, c), rw_sems.at[c, 4])
            recv_wait(w_slot(3 - a, 1, c), w_half(1, c), rw_sems.at[c, 5])

        w_imm(0)
        w_fwd(0)
        w_imm(1)

        recv_wait(x_slot(b, 0), x_half(0), rx_sems.at[2])
        recv_wait(x_slot(jnp.bitwise_xor(b, 1), 0), x_half(0), rx_sems.at[3])
        recv_wait(x_slot(a, 1), x_half(1), rx_sems.at[4])
        recv_wait(x_slot(3 - a, 1), x_half(1), rx_sems.at[5])
        cx.wait()
        cxv = pltpu.make_async_copy(xg_ref, xv, local_sems.at[2])
        cxv.start()

        w_complete(0)
        cw.wait()
        cxv.wait()

        def wv_dma(c, buf):
            return pltpu.make_async_copy(
                wg_ref.at[:, pl.ds(c * cw_, cw_)], wv.at[buf], wv_sems.at[buf])

        def yv_dma(c, i, buf):
            return pltpu.make_async_copy(
                yv.at[buf],
                y_ref.at[pl.ds(i * BM, BM), pl.ds(c * cw_, cw_)],
                yv_sems.at[buf])

        scale = s_ref[0, 0]

        def dot_into(buf, cbuf, i):
            yv[buf] = lax.dot_general(
                xv[pl.ds(i * BM, BM), :], wv[cbuf],
                (((1,), (0,)), ((), ())),
                preferred_element_type=jnp.float32) * scale

        def pair_body(c, p):
            g0 = c * sub_per_chunk + 2 * p
            cbuf = c % 2

            @pl.when(g0 >= 2)
            def _():
                yv_dma(lax.div(g0 - 2, sub_per_chunk),
                       lax.rem(g0 - 2, sub_per_chunk), 0).wait()

            dot_into(0, cbuf, 2 * p)
            yv_dma(c, 2 * p, 0).start()

            @pl.when(g0 >= 2)
            def _():
                yv_dma(lax.div(g0 - 1, sub_per_chunk),
                       lax.rem(g0 - 1, sub_per_chunk), 1).wait()

            dot_into(1, cbuf, 2 * p + 1)
            yv_dma(c, 2 * p + 1, 1).start()

        wv_dma(0, 0).start()
        for c in range(N_CHUNK):
            if c + 1 < N_CHUNK:
                w_fwd(c + 1)
            if c + 2 < N_CHUNK:
                w_imm(c + 2)

            wv_dma(c, c % 2).wait()

            def _pair(p, cr, c=c):
                pair_body(c, p)
                return cr

            lax.fori_loop(0, sub_per_chunk // 2, _pair, 0)

            if c + 1 < N_CHUNK:
                w_complete(c + 1)
                wv_dma(c + 1, (c + 1) % 2).start()

        yv_dma(N_CHUNK - 1, sub_per_chunk - 2, 0).wait()
        yv_dma(N_CHUNK - 1, sub_per_chunk - 1, 1).wait()
        for s in sends:
            s.wait_send()

    return body


def kernel(x, w_mat, scale_x, scale_w):
    m, k_sh = x.shape
    _, n = w_mat.shape
    k = k_sh * N_DEV

    x8 = x.astype(jnp.float8_e5m2)
    w8 = w_mat.astype(jnp.float8_e5m2)
    scale = (scale_x[0] * scale_w[0]).reshape(1, 1)

    y, _, _ = pl.pallas_call(
        _make_fused_body(m, k_sh, n),
        out_shape=[
            jax.ShapeDtypeStruct((m, n), jnp.float32),
            jax.ShapeDtypeStruct((k, n), jnp.float8_e5m2),
            jax.ShapeDtypeStruct((m, k), jnp.float8_e5m2),
        ],
        in_specs=[
            pl.BlockSpec(memory_space=pl.ANY),
            pl.BlockSpec(memory_space=pl.ANY),
            pl.BlockSpec(memory_space=pltpu.SMEM),
        ],
        out_specs=[
            pl.BlockSpec(memory_space=pl.ANY),
            pl.BlockSpec(memory_space=pl.ANY),
            pl.BlockSpec(memory_space=pl.ANY),
        ],
        scratch_shapes=[
            pltpu.VMEM((m, k), jnp.float8_e5m2),
            pltpu.VMEM((2, k, n // N_CHUNK), jnp.float8_e5m2),
            pltpu.VMEM((2, BM, n // N_CHUNK), jnp.float32),
            pltpu.SemaphoreType.DMA((3,)),
            pltpu.SemaphoreType.DMA((2,)),
            pltpu.SemaphoreType.DMA((2,)),
            pltpu.SemaphoreType.DMA((6,)),
            pltpu.SemaphoreType.DMA((6,)),
            pltpu.SemaphoreType.DMA((N_CHUNK, 6)),
            pltpu.SemaphoreType.DMA((N_CHUNK, 6)),
        ],
        compiler_params=pltpu.CompilerParams(
            collective_id=0, vmem_limit_bytes=56 * 1024 * 1024),
    )(x8, w8, scale)
    return y


# device time: 381341 ns/iter; 1.7451x vs baseline; 1.7451x over previous
import jax
import jax.numpy as jnp
from jax import lax
from jax.experimental import pallas as pl
from jax.experimental.pallas import tpu as pltpu

N_DEV = 4


def _make_ag_body(m, k_sh, n):
    h = k_sh // 2

    def body(x_ref, w_ref, xg_ref, wg_ref,
             local_sems, sx_sems, rx_sems, sw_sems, rw_sems):
        me = lax.axis_index("i")
        a = jnp.bitwise_xor(me, 1)
        b = 3 - me

        def x_slot(q, half):
            return xg_ref.at[:, pl.ds((2 * q + half) * h, h)]

        def w_slot(q, half):
            return wg_ref.at[pl.ds((2 * q + half) * h, h), :]

        def x_half(half):
            return x_ref.at[:, pl.ds(half * h, h)]

        def w_half(half):
            return w_ref.at[pl.ds(half * h, h), :]

        def start(src, dst, ssem, rsem, dev):
            c = pltpu.make_async_remote_copy(
                src_ref=src, dst_ref=dst, send_sem=ssem, recv_sem=rsem,
                device_id=(dev,), device_id_type=pl.DeviceIdType.MESH)
            c.start()
            return c

        def recv_wait(dst, dummy_src, rsem):
            pltpu.make_async_remote_copy(
                src_ref=dummy_src, dst_ref=dst,
                send_sem=rsem, recv_sem=rsem,
                device_id=(me,),
                device_id_type=pl.DeviceIdType.MESH).wait_recv()

        barrier = pltpu.get_barrier_semaphore()
        for p in (a, b):
            pl.semaphore_signal(barrier, inc=1, device_id=(p,),
                                device_id_type=pl.DeviceIdType.MESH)
        pl.semaphore_wait(barrier, 2)

        cx = pltpu.make_async_copy(
            x_ref, xg_ref.at[:, pl.ds(me * k_sh, k_sh)], local_sems.at[0])
        cw = pltpu.make_async_copy(
            w_ref, wg_ref.at[pl.ds(me * k_sh, k_sh), :], local_sems.at[1])
        cx.start()
        cw.start()

        sends = []
        for s_sems, src, slot in ((sx_sems, x_half, x_slot),
                                  (sw_sems, w_half, w_slot)):
            r_sems = rx_sems if s_sems is sx_sems else rw_sems
            sends.append(start(src(0), slot(me, 0), s_sems.at[0], r_sems.at[0], a))
            sends.append(start(src(1), slot(me, 1), s_sems.at[1], r_sems.at[1], b))
            sends.append(start(src(0), slot(me, 0), s_sems.at[2], r_sems.at[2], b))
            sends.append(start(src(1), slot(me, 1), s_sems.at[4], r_sems.at[4], a))

        recv_wait(x_slot(a, 0), x_half(0), rx_sems.at[0])
        sends.append(start(x_slot(a, 0), x_slot(a, 0), sx_sems.at[3],
                           rx_sems.at[3], b))
        recv_wait(x_slot(b, 1), x_half(1), rx_sems.at[1])
        sends.append(start(x_slot(b, 1), x_slot(b, 1), sx_sems.at[5],
                           rx_sems.at[5], a))
        recv_wait(w_slot(a, 0), w_half(0), rw_sems.at[0])
        sends.append(start(w_slot(a, 0), w_slot(a, 0), sw_sems.at[3],
                           rw_sems.at[3], b))
        recv_wait(w_slot(b, 1), w_half(1), rw_sems.at[1])
        sends.append(start(w_slot(b, 1), w_slot(b, 1), sw_sems.at[5],
                           rw_sems.at[5], a))

        recv_wait(x_slot(b, 0), x_half(0), rx_sems.at[2])
        recv_wait(x_slot(jnp.bitwise_xor(b, 1), 0), x_half(0), rx_sems.at[3])
        recv_wait(x_slot(a, 1), x_half(1), rx_sems.at[4])
        recv_wait(x_slot(3 - a, 1), x_half(1), rx_sems.at[5])
        recv_wait(w_slot(b, 0), w_half(0), rw_sems.at[2])
        recv_wait(w_slot(jnp.bitwise_xor(b, 1), 0), w_half(0), rw_sems.at[3])
        recv_wait(w_slot(a, 1), w_half(1), rw_sems.at[4])
        recv_wait(w_slot(3 - a, 1), w_half(1), rw_sems.at[5])

        for c in sends:
            c.wait_send()
        cx.wait()
        cw.wait()

    return body


def _gemm_body(xg_ref, wg_ref, s_ref, o_ref):
    acc = lax.dot_general(
        xg_ref[...], wg_ref[...], (((1,), (0,)), ((), ())),
        preferred_element_type=jnp.float32)
    o_ref[...] = acc * s_ref[0, 0]


def kernel(x, w_mat, scale_x, scale_w):
    m, k_sh = x.shape
    _, n = w_mat.shape
    k = k_sh * N_DEV

    x8 = x.astype(jnp.float8_e5m2)
    w8 = w_mat.astype(jnp.float8_e5m2)

    xg, wg = pl.pallas_call(
        _make_ag_body(m, k_sh, n),
        out_shape=[
            jax.ShapeDtypeStruct((m, k), jnp.float8_e5m2),
            jax.ShapeDtypeStruct((k, n), jnp.float8_e5m2),
        ],
        in_specs=[
            pl.BlockSpec(memory_space=pl.ANY),
            pl.BlockSpec(memory_space=pl.ANY),
        ],
        out_specs=[
            pl.BlockSpec(memory_space=pl.ANY),
            pl.BlockSpec(memory_space=pl.ANY),
        ],
        scratch_shapes=[
            pltpu.SemaphoreType.DMA((2,)),
            pltpu.SemaphoreType.DMA((6,)),
            pltpu.SemaphoreType.DMA((6,)),
            pltpu.SemaphoreType.DMA((6,)),
            pltpu.SemaphoreType.DMA((6,)),
        ],
        compiler_params=pltpu.CompilerParams(collective_id=0),
    )(x8, w8)

    scale = (scale_x[0] * scale_w[0]).reshape(1, 1)

    bm, bn = 1024, 2048
    y = pl.pallas_call(
        _gemm_body,
        grid=(m // bm, n // bn),
        out_shape=jax.ShapeDtypeStruct((m, n), jnp.float32),
        in_specs=[
            pl.BlockSpec((bm, k), lambda i, j: (i, 0)),
            pl.BlockSpec((k, bn), lambda i, j: (0, j)),
            pl.BlockSpec((1, 1), lambda i, j: (0, 0),
                         memory_space=pltpu.SMEM),
        ],
        out_specs=pl.BlockSpec((bm, bn), lambda i, j: (i, j)),
        compiler_params=pltpu.CompilerParams(
            dimension_semantics=("parallel", "parallel")),
    )(xg, wg, scale)
    return y
